# baseline (speedup 1.0000x reference)
import functools

import jax
import jax.numpy as jnp
from jax import lax
from jax.experimental import pallas as pl
from jax.experimental.pallas import tpu as pltpu
from jax.experimental.pallas import tpu_sc as plsc

HIDDEN = 64
LANES = 16
TC_BLK = 1024


@functools.cache
def _build_gather(B: int):
    info = plsc.get_sparse_core_info()
    NC, NS = info.num_cores, info.num_subcores
    NW = NC * NS
    assert B % (8 * NW) == 0
    b_per_w = B // NW
    chunk = b_per_w // 2
    mesh = plsc.VectorSubcoreMesh(core_axis_name="c", subcore_axis_name="s")

    @functools.partial(
        pl.kernel,
        mesh=mesh,
        out_type=(
            jax.ShapeDtypeStruct((B, 2 * HIDDEN), jnp.float32),
            jax.ShapeDtypeStruct((B, 2 * HIDDEN), jnp.float32),
        ),
        scratch_types=[
            pltpu.VMEM((b_per_w,), jnp.int32),
            pltpu.VMEM((b_per_w,), jnp.int32),
            pltpu.VMEM((b_per_w,), jnp.int32),
            pltpu.VMEM((b_per_w,), jnp.int32),
            pltpu.VMEM((b_per_w // 2, 2 * HIDDEN), jnp.float32),
            pltpu.VMEM((b_per_w // 2, 2 * HIDDEN), jnp.float32),
            pltpu.SemaphoreType.DMA,
            pltpu.SemaphoreType.DMA,
        ],
    )
    def pair_gather(uid_hbm, iid_hbm, utab_hbm, itab_hbm, gu_hbm, gi_hbm,
                    uidx_v, iidx_v, uprow_v, iprow_v, gu_v, gi_v, usem, isem):
        wid = lax.axis_index("s") * NC + lax.axis_index("c")
        base = wid * b_per_w

        pltpu.sync_copy(uid_hbm.at[pl.ds(base, b_per_w)], uidx_v)
        pltpu.sync_copy(iid_hbm.at[pl.ds(base, b_per_w)], iidx_v)

        def split_body(j, _):
            o = j * LANES
            uprow_v[pl.ds(o, LANES)] = lax.shift_right_logical(
                uidx_v[pl.ds(o, LANES)], 1)
            iprow_v[pl.ds(o, LANES)] = lax.shift_right_logical(
                iidx_v[pl.ds(o, LANES)], 1)
            return _

        lax.fori_loop(0, b_per_w // LANES, split_body, None)

        for k in range(2):
            off = k * chunk
            ucp = pltpu.async_copy(
                utab_hbm.at[uprow_v.at[pl.ds(off, chunk)]], gu_v, usem)
            icp = pltpu.async_copy(
                itab_hbm.at[iprow_v.at[pl.ds(off, chunk)]], gi_v, isem)
            ucp.wait()
            pltpu.sync_copy(gu_v, gu_hbm.at[pl.ds(base + off, chunk)])
            icp.wait()
            pltpu.sync_copy(gi_v, gi_hbm.at[pl.ds(base + off, chunk)])

    return pair_gather


def _select_concat_body(uid_ref, iid_ref, gu_ref, gi_ref, out_ref):
    pu = (uid_ref[0, 0] & 1)[:, None].astype(jnp.bool_)
    pi = (iid_ref[0, 0] & 1)[:, None].astype(jnp.bool_)
    gu = gu_ref[...]
    gi = gi_ref[...]
    u_sel = jnp.where(pu, gu[:, HIDDEN:], gu[:, :HIDDEN])
    i_sel = jnp.where(pi, gi[:, HIDDEN:], gi[:, :HIDDEN])
    out_ref[...] = jnp.concatenate([u_sel, i_sel], axis=-1)


@functools.cache
def _build_select(B: int):
    nblk = B // TC_BLK
    return pl.pallas_call(
        _select_concat_body,
        grid=(nblk,),
        in_specs=[
            pl.BlockSpec((1, 1, TC_BLK), lambda b: (b, 0, 0)),
            pl.BlockSpec((1, 1, TC_BLK), lambda b: (b, 0, 0)),
            pl.BlockSpec((TC_BLK, 2 * HIDDEN), lambda b: (b, 0)),
            pl.BlockSpec((TC_BLK, 2 * HIDDEN), lambda b: (b, 0)),
        ],
        out_specs=pl.BlockSpec((TC_BLK, 2 * HIDDEN), lambda b: (b, 0)),
        out_shape=jax.ShapeDtypeStruct((B, 2 * HIDDEN), jnp.float32),
    )


def kernel(user_id, item_id, user_emb, item_emb):
    B = user_id.shape[0]
    u128 = user_emb.reshape(user_emb.shape[0] // 2, 2 * HIDDEN)
    i128 = item_emb.reshape(item_emb.shape[0] // 2, 2 * HIDDEN)
    gu, gi = _build_gather(B)(user_id, item_id, u128, i128)
    uid2 = user_id.reshape(B // TC_BLK, 1, TC_BLK)
    iid2 = item_id.reshape(B // TC_BLK, 1, TC_BLK)
    return _build_select(B)(uid2, iid2, gu, gi)


# speedup vs baseline: 1.0731x; 1.0731x over previous
import functools

import jax
import jax.numpy as jnp
from jax import lax
from jax.experimental import pallas as pl
from jax.experimental.pallas import tpu as pltpu
from jax.experimental.pallas import tpu_sc as plsc

HIDDEN = 64
LANES = 16
TC_BLK = 1024


@functools.cache
def _build_gather(B: int):
    info = plsc.get_sparse_core_info()
    NC, NS = info.num_cores, info.num_subcores
    NW = NC * NS
    assert B % (8 * NW) == 0
    b_per_w = B // NW
    chunk = b_per_w // 2
    mesh = plsc.VectorSubcoreMesh(core_axis_name="c", subcore_axis_name="s")

    @functools.partial(
        pl.kernel,
        mesh=mesh,
        out_type=(
            jax.ShapeDtypeStruct((B, 2 * HIDDEN), jnp.float32),
            jax.ShapeDtypeStruct((B, 2 * HIDDEN), jnp.float32),
        ),
        scratch_types=[
            pltpu.VMEM((b_per_w,), jnp.int32),
            pltpu.VMEM((b_per_w,), jnp.int32),
            pltpu.VMEM((b_per_w // 2, 2 * HIDDEN), jnp.float32),
            pltpu.VMEM((b_per_w // 2, 2 * HIDDEN), jnp.float32),
            pltpu.SemaphoreType.DMA,
            pltpu.SemaphoreType.DMA,
        ],
    )
    def row_gather(uid_hbm, iid_hbm, utab_hbm, itab_hbm, gu_hbm, gi_hbm,
                   uidx_v, iidx_v, gu_v, gi_v, usem, isem):
        wid = lax.axis_index("s") * NC + lax.axis_index("c")
        base = wid * b_per_w

        pltpu.sync_copy(uid_hbm.at[pl.ds(base, b_per_w)], uidx_v)
        pltpu.sync_copy(iid_hbm.at[pl.ds(base, b_per_w)], iidx_v)

        for k in range(2):
            off = k * chunk
            ucp = pltpu.async_copy(
                utab_hbm.at[uidx_v.at[pl.ds(off, chunk)]], gu_v, usem)
            icp = pltpu.async_copy(
                itab_hbm.at[iidx_v.at[pl.ds(off, chunk)]], gi_v, isem)
            ucp.wait()
            pltpu.sync_copy(gu_v, gu_hbm.at[pl.ds(base + off, chunk)])
            icp.wait()
            pltpu.sync_copy(gi_v, gi_hbm.at[pl.ds(base + off, chunk)])

    return row_gather


def _concat_body(gu_ref, gi_ref, out_ref):
    out_ref[...] = jnp.concatenate(
        [gu_ref[:, :HIDDEN], gi_ref[:, :HIDDEN]], axis=-1)


@functools.cache
def _build_concat(B: int):
    nblk = B // TC_BLK
    return pl.pallas_call(
        _concat_body,
        grid=(nblk,),
        in_specs=[
            pl.BlockSpec((TC_BLK, 2 * HIDDEN), lambda b: (b, 0)),
            pl.BlockSpec((TC_BLK, 2 * HIDDEN), lambda b: (b, 0)),
        ],
        out_specs=pl.BlockSpec((TC_BLK, 2 * HIDDEN), lambda b: (b, 0)),
        out_shape=jax.ShapeDtypeStruct((B, 2 * HIDDEN), jnp.float32),
    )


def kernel(user_id, item_id, user_emb, item_emb):
    B = user_id.shape[0]
    u128 = jnp.pad(user_emb, ((0, 0), (0, HIDDEN)))
    i128 = jnp.pad(item_emb, ((0, 0), (0, HIDDEN)))
    gu, gi = _build_gather(B)(user_id, item_id, u128, i128)
    return _build_concat(B)(gu, gi)


# speedup vs baseline: 1.0873x; 1.0133x over previous
import functools

import jax
import jax.numpy as jnp
from jax import lax
from jax.experimental import pallas as pl
from jax.experimental.pallas import tpu as pltpu
from jax.experimental.pallas import tpu_sc as plsc

HIDDEN = 64
LANES = 16


@functools.cache
def _build_gather(B: int):
    info = plsc.get_sparse_core_info()
    NC, NS = info.num_cores, info.num_subcores
    NW = NC * NS
    assert B % (8 * NW) == 0
    b_per_w = B // NW
    chunk = b_per_w // 2
    mesh = plsc.VectorSubcoreMesh(core_axis_name="c", subcore_axis_name="s")

    @functools.partial(
        pl.kernel,
        mesh=mesh,
        out_type=jax.ShapeDtypeStruct((B, 2 * HIDDEN), jnp.float32),
        scratch_types=[
            pltpu.VMEM((b_per_w,), jnp.int32),
            pltpu.VMEM((b_per_w,), jnp.int32),
            pltpu.VMEM((b_per_w // 2, 2 * HIDDEN), jnp.float32),
            pltpu.VMEM((b_per_w // 2, 2 * HIDDEN), jnp.float32),
            pltpu.SemaphoreType.DMA,
            pltpu.SemaphoreType.DMA,
        ],
    )
    def gather_concat(uid_hbm, iid_hbm, utab_hbm, itab_hbm, out_hbm,
                      uidx_v, iidx_v, gu_v, gi_v, usem, isem):
        wid = lax.axis_index("s") * NC + lax.axis_index("c")
        base = wid * b_per_w

        pltpu.sync_copy(uid_hbm.at[pl.ds(base, b_per_w)], uidx_v)
        pltpu.sync_copy(iid_hbm.at[pl.ds(base, b_per_w)], iidx_v)

        def merge_row(r, _):
            for c in range(0, HIDDEN, LANES):
                gu_v[r, pl.ds(HIDDEN + c, LANES)] = gi_v[r, pl.ds(c, LANES)]
            return _

        for k in range(2):
            off = k * chunk
            ucp = pltpu.async_copy(
                utab_hbm.at[uidx_v.at[pl.ds(off, chunk)]], gu_v, usem)
            icp = pltpu.async_copy(
                itab_hbm.at[iidx_v.at[pl.ds(off, chunk)]], gi_v, isem)
            ucp.wait()
            icp.wait()
            lax.fori_loop(0, chunk, merge_row, None)
            pltpu.sync_copy(gu_v, out_hbm.at[pl.ds(base + off, chunk)])

    return gather_concat


def kernel(user_id, item_id, user_emb, item_emb):
    B = user_id.shape[0]
    u128 = jnp.pad(user_emb, ((0, 0), (0, HIDDEN)))
    i128 = jnp.pad(item_emb, ((0, 0), (0, HIDDEN)))
    return _build_gather(B)(user_id, item_id, u128, i128)


# speedup vs baseline: 1.2346x; 1.1354x over previous
import functools

import jax
import jax.numpy as jnp
from jax import lax
from jax.experimental import pallas as pl
from jax.experimental.pallas import tpu as pltpu
from jax.experimental.pallas import tpu_sc as plsc

HIDDEN = 64
LANES = 16


@functools.cache
def _build_gather(B: int):
    info = plsc.get_sparse_core_info()
    NC, NS = info.num_cores, info.num_subcores
    NW = NC * NS
    assert B % (8 * NW) == 0
    b_per_w = B // NW
    chunk = b_per_w // 2
    mesh = plsc.VectorSubcoreMesh(core_axis_name="c", subcore_axis_name="s")

    @functools.partial(
        pl.kernel,
        mesh=mesh,
        out_type=jax.ShapeDtypeStruct((B, 2 * HIDDEN), jnp.float32),
        scratch_types=[
            pltpu.VMEM((b_per_w,), jnp.int32),
            pltpu.VMEM((b_per_w,), jnp.int32),
            pltpu.VMEM((b_per_w // 2, 2 * HIDDEN), jnp.float32),
            pltpu.VMEM((b_per_w // 2, 2 * HIDDEN), jnp.float32),
            pltpu.SemaphoreType.DMA,
            pltpu.SemaphoreType.DMA,
        ],
    )
    def gather_concat(uid_hbm, iid_hbm, ctab_hbm, out_hbm,
                      uidx_v, iidx_v, gu_v, gi_v, usem, isem):
        wid = lax.axis_index("s") * NC + lax.axis_index("c")
        base = wid * b_per_w

        pltpu.sync_copy(uid_hbm.at[pl.ds(base, b_per_w)], uidx_v)
        pltpu.sync_copy(iid_hbm.at[pl.ds(base, b_per_w)], iidx_v)

        def merge_row(r, _):
            for c in range(0, HIDDEN, LANES):
                gu_v[r, pl.ds(HIDDEN + c, LANES)] = gi_v[r, pl.ds(HIDDEN + c, LANES)]
            return _

        for k in range(2):
            off = k * chunk
            ucp = pltpu.async_copy(
                ctab_hbm.at[uidx_v.at[pl.ds(off, chunk)]], gu_v, usem)
            icp = pltpu.async_copy(
                ctab_hbm.at[iidx_v.at[pl.ds(off, chunk)]], gi_v, isem)
            ucp.wait()
            icp.wait()
            lax.fori_loop(0, chunk, merge_row, None)
            pltpu.sync_copy(gu_v, out_hbm.at[pl.ds(base + off, chunk)])

    return gather_concat


def kernel(user_id, item_id, user_emb, item_emb):
    B = user_id.shape[0]
    ctab = jnp.concatenate([user_emb, item_emb], axis=1)
    return _build_gather(B)(user_id, item_id, ctab)
